# baseline (speedup 1.0000x reference)
import functools

import jax
import jax.numpy as jnp
from jax.experimental import pallas as pl
from jax.experimental.pallas import tpu as pltpu


def _se_kernel(x_ref, w1t_ref, w2t_ref, b2_ref, o_ref, *, inv_hw):
    x = x_ref[...]
    pooled = jnp.sum(x, axis=2) * inv_hw
    h = jnp.maximum(
        jnp.dot(pooled, w1t_ref[...],
                preferred_element_type=jnp.float32), 0.0)
    z = jnp.dot(h, w2t_ref[...],
                preferred_element_type=jnp.float32)
    gate = jax.nn.sigmoid(z + b2_ref[...])
    o_ref[...] = x * gate[:, :, None]


def kernel(x, w1, w2, b2):
    N, C, H, W = x.shape
    Cr = w1.shape[0]
    HW = H * W

    x_flat = x.reshape(N, C, HW)
    w1t = w1.T
    w2t = w2.T
    b2_row = b2.reshape(1, C)

    nb = 4
    while N % nb:
        nb //= 2
    steps = N // nb
    blk = (nb, C, HW)

    body = functools.partial(_se_kernel, inv_hw=1.0 / HW)
    out_flat = pl.pallas_call(
        body,
        out_shape=jax.ShapeDtypeStruct((N, C, HW), x.dtype),
        grid=(steps,),
        in_specs=[
            pl.BlockSpec(blk, lambda n: (n, 0, 0)),
            pl.BlockSpec(w1t.shape, lambda n: (0, 0)),
            pl.BlockSpec(w2t.shape, lambda n: (0, 0)),
            pl.BlockSpec(b2_row.shape, lambda n: (0, 0)),
        ],
        out_specs=pl.BlockSpec(blk, lambda n: (n, 0, 0)),
        compiler_params=pltpu.CompilerParams(
            dimension_semantics=("parallel",),
            vmem_limit_bytes=64 << 20,
        ),
        cost_estimate=pl.CostEstimate(
            flops=int(N * (2 * C * HW + 4 * C * Cr + 3 * C)),
            transcendentals=int(N * C),
            bytes_accessed=int(2 * N * C * HW * 4),
        ),
    )(x_flat, w1t, w2t, b2_row)
    return out_flat.reshape(N, C, H, W)


# speedup vs baseline: 1.0614x; 1.0614x over previous
import functools

import jax
import jax.numpy as jnp
from jax.experimental import pallas as pl
from jax.experimental.pallas import tpu as pltpu


def _se_kernel(x_ref, w1t_ref, w2t_ref, b2_ref, o_ref, *, inv_hw):
    x = x_ref[...]
    pooled = jnp.sum(x, axis=2) * inv_hw
    h = jnp.maximum(
        jnp.dot(pooled, w1t_ref[...],
                preferred_element_type=jnp.float32), 0.0)
    z = jnp.dot(h, w2t_ref[...],
                preferred_element_type=jnp.float32)
    gate = jax.nn.sigmoid(z + b2_ref[...])
    o_ref[...] = x * gate[:, :, None]


def kernel(x, w1, w2, b2):
    N, C, H, W = x.shape
    Cr = w1.shape[0]
    HW = H * W

    x_flat = x.reshape(N, C, HW)
    w1t = w1.T
    w2t = w2.T
    b2_row = b2.reshape(1, C)

    nb = 16
    while N % nb:
        nb //= 2
    steps = N // nb
    blk = (nb, C, HW)

    body = functools.partial(_se_kernel, inv_hw=1.0 / HW)
    out_flat = pl.pallas_call(
        body,
        out_shape=jax.ShapeDtypeStruct((N, C, HW), x.dtype),
        grid=(steps,),
        in_specs=[
            pl.BlockSpec(blk, lambda n: (n, 0, 0)),
            pl.BlockSpec(w1t.shape, lambda n: (0, 0)),
            pl.BlockSpec(w2t.shape, lambda n: (0, 0)),
            pl.BlockSpec(b2_row.shape, lambda n: (0, 0)),
        ],
        out_specs=pl.BlockSpec(blk, lambda n: (n, 0, 0)),
        compiler_params=pltpu.CompilerParams(
            dimension_semantics=("parallel",),
            vmem_limit_bytes=64 << 20,
        ),
        cost_estimate=pl.CostEstimate(
            flops=int(N * (2 * C * HW + 4 * C * Cr + 3 * C)),
            transcendentals=int(N * C),
            bytes_accessed=int(2 * N * C * HW * 4),
        ),
    )(x_flat, w1t, w2t, b2_row)
    return out_flat.reshape(N, C, H, W)
